# baseline (speedup 1.0000x reference)
import functools

import jax
import jax.numpy as jnp
from jax import lax
from jax.experimental import pallas as pl
from jax.experimental.pallas import tpu as pltpu
from jax.experimental.pallas import tpu_sc as plsc

_NUM_WORKERS = 32
_CHUNK = 400
_SC_FRACTION_NUM = 90
_SC_FRACTION_DEN = 100


def _stats_kernel(w_ref, k_ref, out_ref):
    wv = w_ref[...]
    m = jnp.max(wv)
    d = jnp.sum(jnp.exp(wv - m))
    coeff = jnp.round(k_ref[0, 0]) / (jnp.float32(wv.size) * d)
    out_ref[...] = jnp.stack([jnp.full((16,), m), jnp.full((16,), coeff)])


def _wsum_tc_kernel(stats_ref, w_ref, x_ref, out_ref):
    i = pl.program_id(0)
    m = stats_ref[0, 0]
    coeff = stats_ref[1, 0]
    e = jnp.exp(w_ref[0] - m) * coeff
    part = jax.lax.dot_general(
        e, x_ref[...], (((1,), (0,)), ((), ())),
        preferred_element_type=jnp.float32)

    @pl.when(i == 0)
    def _init():
        out_ref[...] = jnp.zeros_like(out_ref)

    out_ref[...] += part


def _combine_kernel(tc_ref, sc_ref, out_ref):
    out_ref[...] = tc_ref[...] + jnp.sum(sc_ref[...], axis=0, keepdims=True)


def _splat16(v, idx):
    return lax.gather(
        v, idx[:, None],
        dimension_numbers=lax.GatherDimensionNumbers(
            offset_dims=(), collapsed_slice_dims=(0,), start_index_map=(0,)),
        slice_sizes=(1,),
        mode=lax.GatherScatterMode.PROMISE_IN_BOUNDS)


def _make_sc_wsum(n_sc, t):
    c = _CHUNK
    cw = c * t
    rows_tile = n_sc // _NUM_WORKERS
    nc = rows_tile // c
    mesh = plsc.VectorSubcoreMesh(core_axis_name="c", subcore_axis_name="s")

    @functools.partial(
        pl.kernel,
        out_type=jax.ShapeDtypeStruct((_NUM_WORKERS * t,), jnp.float32),
        mesh=mesh,
        scratch_types=[
            pltpu.VMEM((c, t), jnp.float32),
            pltpu.VMEM((c, t), jnp.float32),
            pltpu.VMEM((c,), jnp.float32),
            pltpu.VMEM((c,), jnp.float32),
            pltpu.VMEM((32,), jnp.float32),
            pltpu.VMEM((t,), jnp.float32),
            pltpu.SemaphoreType.DMA,
            pltpu.SemaphoreType.DMA,
            pltpu.SemaphoreType.DMA,
            pltpu.SemaphoreType.DMA,
        ],
    )
    def sc_fn(stats_hbm, w_hbm, x_hbm, out_hbm,
              xa, xb, wa, wb, sb, ob, sxa, sxb, swa, swb):
        cid = lax.axis_index("c")
        sid = lax.axis_index("s")
        wid = sid * 2 + cid
        base = wid * rows_tile

        pltpu.sync_copy(stats_hbm, sb)
        mv = sb[pl.ds(0, 16)]
        cv = sb[pl.ds(16, 16)]

        def start_chunk(j, xbuf, wbuf, sx, sw):
            r0 = base + j * c
            pltpu.async_copy(x_hbm.at[pl.ds(r0, c), :], xbuf, sx)
            pltpu.async_copy(w_hbm.at[pl.ds(r0, c)], wbuf, sw)

        def wait_chunk(xbuf, wbuf, sx, sw):
            pltpu.make_async_copy(x_hbm.at[pl.ds(0, c), :], xbuf, sx).wait()
            pltpu.make_async_copy(w_hbm.at[pl.ds(0, c)], wbuf, sw).wait()

        def compute(xbuf, wbuf, acc):
            def grp(g, acc):
                wv = wbuf[pl.ds(g * 16, 16)]
                e = jnp.exp(wv - mv) * cv
                a0, a1, a2, a3 = acc
                for j in range(16):
                    ej = _splat16(e, jnp.full((16,), j, jnp.int32))
                    row = g * 16 + j
                    a0 = a0 + ej * xbuf[row, pl.ds(0, 16)]
                    a1 = a1 + ej * xbuf[row, pl.ds(16, 16)]
                    a2 = a2 + ej * xbuf[row, pl.ds(32, 16)]
                    a3 = a3 + ej * xbuf[row, pl.ds(48, 16)]
                return a0, a1, a2, a3
            return lax.fori_loop(0, c // 16, grp, acc)

        start_chunk(0, xa, wa, sxa, swa)
        start_chunk(1, xb, wb, sxb, swb)
        z = jnp.zeros((16,), jnp.float32)
        acc0 = (z, z, z, z)

        def body(i, acc):
            j0 = 2 * i
            wait_chunk(xa, wa, sxa, swa)
            acc = compute(xa, wa, acc)

            @pl.when(j0 + 2 < nc)
            def _():
                start_chunk(j0 + 2, xa, wa, sxa, swa)

            wait_chunk(xb, wb, sxb, swb)
            acc = compute(xb, wb, acc)

            @pl.when(j0 + 3 < nc)
            def _():
                start_chunk(j0 + 3, xb, wb, sxb, swb)

            return acc

        acc = lax.fori_loop(0, nc // 2, body, acc0)
        ob[pl.ds(0, 16)] = acc[0]
        ob[pl.ds(16, 16)] = acc[1]
        ob[pl.ds(32, 16)] = acc[2]
        ob[pl.ds(48, 16)] = acc[3]
        pltpu.sync_copy(ob, out_hbm.at[pl.ds(wid * t, t)])

    return sc_fn


def _pick_block(n):
    for b in (8000, 10000, 5000, 4096, 4000, 2048, 2000, 1000):
        if n % b == 0:
            return b
    return n


def _tc_wsum(stats, w, x, first_block, num_blocks, b):
    n, t = x.shape
    g_total = n // b
    return pl.pallas_call(
        _wsum_tc_kernel,
        grid=(num_blocks,),
        out_shape=jax.ShapeDtypeStruct((1, t), jnp.float32),
        in_specs=[
            pl.BlockSpec((2, 16), lambda i: (0, 0)),
            pl.BlockSpec((1, 1, b), lambda i: (first_block + i, 0, 0)),
            pl.BlockSpec((b, t), lambda i: (first_block + i, 0)),
        ],
        out_specs=pl.BlockSpec((1, t), lambda i: (0, 0)),
    )(stats, w.reshape(g_total, 1, b), x)


def kernel(x, w, k_param):
    n, t = x.shape
    b = _pick_block(n)
    rows = 1000 if n % 1000 == 0 else 1

    stats = pl.pallas_call(
        _stats_kernel,
        out_shape=jax.ShapeDtypeStruct((2, 16), jnp.float32),
        in_specs=[
            pl.BlockSpec((n // rows, rows), lambda: (0, 0)),
            pl.BlockSpec((1, 1), lambda: (0, 0)),
        ],
        out_specs=pl.BlockSpec((2, 16), lambda: (0, 0)),
    )(w.reshape(n // rows, rows), k_param.reshape(1, 1))

    stripe = _NUM_WORKERS * _CHUNK
    n_sc = (n * _SC_FRACTION_NUM // _SC_FRACTION_DEN) // stripe * stripe
    while n_sc > 0 and not ((n_sc // stripe) % 2 == 0
                            and (n - n_sc) % b == 0 and n_sc % b == 0):
        n_sc -= stripe
    use_sc = t == 64 and n_sc > 0

    if use_sc:
        sc_part = _make_sc_wsum(n_sc, t)(stats.reshape(32), w, x)
        tc_part = _tc_wsum(stats, w, x, n_sc // b, (n - n_sc) // b, b)
        out = pl.pallas_call(
            _combine_kernel,
            out_shape=jax.ShapeDtypeStruct((1, t), jnp.float32),
            in_specs=[
                pl.BlockSpec((1, t), lambda: (0, 0)),
                pl.BlockSpec((_NUM_WORKERS, t), lambda: (0, 0)),
            ],
            out_specs=pl.BlockSpec((1, t), lambda: (0, 0)),
        )(tc_part, sc_part.reshape(_NUM_WORKERS, t))
    else:
        out = _tc_wsum(stats, w, x, 0, n // b, b)

    return out.reshape(t)


# speedup vs baseline: 1.0515x; 1.0515x over previous
import functools

import jax
import jax.numpy as jnp
from jax import lax
from jax.experimental import pallas as pl
from jax.experimental.pallas import tpu as pltpu
from jax.experimental.pallas import tpu_sc as plsc

_NUM_CORES = 2
_NUM_SUBCORES = 16
_NUM_WORKERS = _NUM_CORES * _NUM_SUBCORES
_CHUNK = 400
_SC_FRACTION_NUM = 64
_SC_FRACTION_DEN = 100


def _stats_kernel(w_ref, k_ref, out_ref):
    wv = w_ref[...]
    m = jnp.max(wv)
    d = jnp.sum(jnp.exp(wv - m))
    coeff = jnp.round(k_ref[0, 0]) / (jnp.float32(wv.size) * d)
    out_ref[...] = jnp.stack([jnp.full((16,), m), jnp.full((16,), coeff)])


def _wsum_tc_grid_kernel(stats_ref, w_ref, x_ref, out_ref):
    i = pl.program_id(0)
    m = stats_ref[0, 0]
    coeff = stats_ref[1, 0]
    e = jnp.exp(w_ref[0] - m) * coeff
    part = jax.lax.dot_general(
        e, x_ref[...], (((1,), (0,)), ((), ())),
        preferred_element_type=jnp.float32)

    @pl.when(i == 0)
    def _init():
        out_ref[...] = jnp.zeros_like(out_ref)

    out_ref[...] += part


def _combine_kernel(tc_ref, sc0_ref, sc1_ref, out_ref):
    out_ref[...] = (tc_ref[...]
                    + jnp.sum(sc0_ref[...], axis=0, keepdims=True)
                    + jnp.sum(sc1_ref[...], axis=0, keepdims=True))


def _splat16(v, idx):
    return lax.gather(
        v, idx[:, None],
        dimension_numbers=lax.GatherDimensionNumbers(
            offset_dims=(), collapsed_slice_dims=(0,), start_index_map=(0,)),
        slice_sizes=(1,),
        mode=lax.GatherScatterMode.PROMISE_IN_BOUNDS)


def _make_sc_wsum(n_sc, t):
    c = _CHUNK
    rows_tile = n_sc // _NUM_WORKERS
    nc = rows_tile // c
    mesh = plsc.VectorSubcoreMesh(core_axis_name="c", subcore_axis_name="s")

    @functools.partial(
        pl.kernel,
        out_type=(
            jax.ShapeDtypeStruct((_NUM_SUBCORES * t,), jnp.float32),
            jax.ShapeDtypeStruct((_NUM_SUBCORES * t,), jnp.float32),
        ),
        mesh=mesh,
        scratch_types=[
            pltpu.VMEM((c, t), jnp.float32),
            pltpu.VMEM((c, t), jnp.float32),
            pltpu.VMEM((c,), jnp.float32),
            pltpu.VMEM((c,), jnp.float32),
            pltpu.VMEM((2, 16), jnp.float32),
            pltpu.VMEM((t,), jnp.float32),
            pltpu.SemaphoreType.DMA,
            pltpu.SemaphoreType.DMA,
            pltpu.SemaphoreType.DMA,
            pltpu.SemaphoreType.DMA,
        ],
    )
    def sc_fn(stats_hbm, w_hbm, x_hbm, out0, out1,
              xa, xb, wa, wb, sb, ob, sxa, sxb, swa, swb):
        cid = lax.axis_index("c")
        sid = lax.axis_index("s")
        wid = sid * _NUM_CORES + cid
        base = wid * rows_tile

        pltpu.sync_copy(stats_hbm, sb)
        mv = sb[0, pl.ds(0, 16)]
        cv = sb[1, pl.ds(0, 16)]

        def start_chunk(j, xbuf, wbuf, sx, sw):
            r0 = base + j * c
            pltpu.async_copy(x_hbm.at[pl.ds(r0, c), :], xbuf, sx)
            pltpu.async_copy(w_hbm.at[pl.ds(r0, c)], wbuf, sw)

        def wait_chunk(xbuf, wbuf, sx, sw):
            pltpu.make_async_copy(x_hbm.at[pl.ds(0, c), :], xbuf, sx).wait()
            pltpu.make_async_copy(w_hbm.at[pl.ds(0, c)], wbuf, sw).wait()

        def compute(xbuf, wbuf, acc):
            def grp(g, acc):
                wv = wbuf[pl.ds(g * 16, 16)]
                e = jnp.exp(wv - mv) * cv
                a0, a1, a2, a3 = acc
                for j in range(16):
                    ej = _splat16(e, jnp.full((16,), j, jnp.int32))
                    row = g * 16 + j
                    a0 = a0 + ej * xbuf[row, pl.ds(0, 16)]
                    a1 = a1 + ej * xbuf[row, pl.ds(16, 16)]
                    a2 = a2 + ej * xbuf[row, pl.ds(32, 16)]
                    a3 = a3 + ej * xbuf[row, pl.ds(48, 16)]
                return a0, a1, a2, a3
            return lax.fori_loop(0, c // 16, grp, acc)

        start_chunk(0, xa, wa, sxa, swa)
        start_chunk(1, xb, wb, sxb, swb)
        z = jnp.zeros((16,), jnp.float32)

        def body(i, acc):
            j0 = 2 * i
            wait_chunk(xa, wa, sxa, swa)
            acc = compute(xa, wa, acc)

            @pl.when(j0 + 2 < nc)
            def _():
                start_chunk(j0 + 2, xa, wa, sxa, swa)

            wait_chunk(xb, wb, sxb, swb)
            acc = compute(xb, wb, acc)

            @pl.when(j0 + 3 < nc)
            def _():
                start_chunk(j0 + 3, xb, wb, sxb, swb)

            return acc

        acc = lax.fori_loop(0, nc // 2, body, (z, z, z, z))
        ob[pl.ds(0, 16)] = acc[0]
        ob[pl.ds(16, 16)] = acc[1]
        ob[pl.ds(32, 16)] = acc[2]
        ob[pl.ds(48, 16)] = acc[3]

        @pl.when(cid == 0)
        def _w0():
            pltpu.sync_copy(ob, out0.at[pl.ds(sid * t, t)])

        @pl.when(cid == 1)
        def _w1():
            pltpu.sync_copy(ob, out1.at[pl.ds(sid * t, t)])

    return sc_fn


def _pick_block(n):
    for b in (8000, 10000, 5000, 4096, 4000, 2048, 2000, 1000):
        if n % b == 0:
            return b
    return n


def _tc_wsum(stats, w, x, first_block, num_blocks, b):
    n, t = x.shape
    g_total = n // b
    return pl.pallas_call(
        _wsum_tc_grid_kernel,
        grid=(num_blocks,),
        out_shape=jax.ShapeDtypeStruct((1, t), jnp.float32),
        in_specs=[
            pl.BlockSpec((2, 16), lambda i: (0, 0)),
            pl.BlockSpec((1, 1, b), lambda i: (first_block + i, 0, 0)),
            pl.BlockSpec((b, t), lambda i: (first_block + i, 0)),
        ],
        out_specs=pl.BlockSpec((1, t), lambda i: (0, 0)),
    )(stats, w.reshape(g_total, 1, b), x)


def kernel(x, w, k_param):
    n, t = x.shape
    b = _pick_block(n)
    rows = 1000 if n % 1000 == 0 else 1

    stats = pl.pallas_call(
        _stats_kernel,
        out_shape=jax.ShapeDtypeStruct((2, 16), jnp.float32),
        in_specs=[
            pl.BlockSpec((n // rows, rows), lambda: (0, 0)),
            pl.BlockSpec((1, 1), lambda: (0, 0)),
        ],
        out_specs=pl.BlockSpec((2, 16), lambda: (0, 0)),
    )(w.reshape(n // rows, rows), k_param.reshape(1, 1))

    stripe = _NUM_WORKERS * _CHUNK
    n_sc = (n * _SC_FRACTION_NUM // _SC_FRACTION_DEN) // stripe * stripe
    while n_sc > 0 and not ((n_sc // stripe) % 2 == 0
                            and (n - n_sc) % b == 0 and n_sc % b == 0):
        n_sc -= stripe
    use_sc = t == 64 and n_sc > 0

    if use_sc:
        sc0, sc1 = _make_sc_wsum(n_sc, t)(stats, w, x)
        tc_part = _tc_wsum(stats, w, x, n_sc // b, (n - n_sc) // b, b)
        out = pl.pallas_call(
            _combine_kernel,
            out_shape=jax.ShapeDtypeStruct((1, t), jnp.float32),
            in_specs=[
                pl.BlockSpec((1, t), lambda: (0, 0)),
                pl.BlockSpec((_NUM_SUBCORES, t), lambda: (0, 0)),
                pl.BlockSpec((_NUM_SUBCORES, t), lambda: (0, 0)),
            ],
            out_specs=pl.BlockSpec((1, t), lambda: (0, 0)),
        )(tc_part, sc0.reshape(_NUM_SUBCORES, t), sc1.reshape(_NUM_SUBCORES, t))
    else:
        out = _tc_wsum(stats, w, x, 0, n // b, b)

    return out.reshape(t)


# speedup vs baseline: 1.0523x; 1.0008x over previous
import functools

import jax
import jax.numpy as jnp
from jax import lax
from jax.experimental import pallas as pl
from jax.experimental.pallas import tpu as pltpu
from jax.experimental.pallas import tpu_sc as plsc

_NUM_CORES = 2
_NUM_SUBCORES = 16
_NUM_WORKERS = _NUM_CORES * _NUM_SUBCORES
_CHUNK = 400
_SC_FRACTION_NUM = 64
_SC_FRACTION_DEN = 100


def _stats_kernel(w_ref, k_ref, out_ref):
    wv = w_ref[...]
    m = jnp.max(wv)
    d = jnp.sum(jnp.exp(wv - m))
    coeff = jnp.round(k_ref[0, 0]) / (jnp.float32(wv.size) * d)
    out_ref[...] = jnp.stack([jnp.full((16,), m), jnp.full((16,), coeff)])


def _wsum_tc_grid_kernel(stats_ref, w_ref, x_ref, out_ref):
    i = pl.program_id(0)
    m = stats_ref[0, 0]
    coeff = stats_ref[1, 0]
    e = jnp.exp(w_ref[0] - m) * coeff
    part = jax.lax.dot_general(
        e, x_ref[...], (((1,), (0,)), ((), ())),
        preferred_element_type=jnp.float32)

    @pl.when(i == 0)
    def _init():
        out_ref[...] = jnp.zeros_like(out_ref)

    out_ref[...] += part


def _combine_kernel(tc_ref, sc0_ref, sc1_ref, out_ref):
    out_ref[...] = (tc_ref[...]
                    + jnp.sum(sc0_ref[...], axis=0, keepdims=True)
                    + jnp.sum(sc1_ref[...], axis=0, keepdims=True))


def _splat16(v, idx):
    return lax.gather(
        v, idx[:, None],
        dimension_numbers=lax.GatherDimensionNumbers(
            offset_dims=(), collapsed_slice_dims=(0,), start_index_map=(0,)),
        slice_sizes=(1,),
        mode=lax.GatherScatterMode.PROMISE_IN_BOUNDS)


def _make_sc_wsum(n_sc, t):
    c = _CHUNK
    rows_tile = n_sc // _NUM_WORKERS
    nc = rows_tile // c
    mesh = plsc.VectorSubcoreMesh(core_axis_name="c", subcore_axis_name="s")

    @functools.partial(
        pl.kernel,
        out_type=(
            jax.ShapeDtypeStruct((_NUM_SUBCORES * t,), jnp.float32),
            jax.ShapeDtypeStruct((_NUM_SUBCORES * t,), jnp.float32),
        ),
        mesh=mesh,
        scratch_types=[
            pltpu.VMEM((c, t), jnp.float32),
            pltpu.VMEM((c, t), jnp.float32),
            pltpu.VMEM((c,), jnp.float32),
            pltpu.VMEM((c,), jnp.float32),
            pltpu.VMEM((2, 16), jnp.float32),
            pltpu.VMEM((t,), jnp.float32),
            pltpu.SemaphoreType.DMA,
            pltpu.SemaphoreType.DMA,
            pltpu.SemaphoreType.DMA,
            pltpu.SemaphoreType.DMA,
        ],
        compiler_params=pltpu.CompilerParams(use_tc_tiling_on_sc=True),
    )
    def sc_fn(stats_hbm, w_hbm, x_hbm, out0, out1,
              xa, xb, wa, wb, sb, ob, sxa, sxb, swa, swb):
        cid = lax.axis_index("c")
        sid = lax.axis_index("s")
        wid = sid * _NUM_CORES + cid
        base = wid * rows_tile

        pltpu.sync_copy(stats_hbm, sb)
        mv = sb[0, pl.ds(0, 16)]
        cv = sb[1, pl.ds(0, 16)]

        def start_chunk(j, xbuf, wbuf, sx, sw):
            r0 = base + j * c
            pltpu.async_copy(x_hbm.at[pl.ds(r0, c), :], xbuf, sx)
            pltpu.async_copy(w_hbm.at[pl.ds(r0, c)], wbuf, sw)

        def wait_chunk(xbuf, wbuf, sx, sw):
            pltpu.make_async_copy(x_hbm.at[pl.ds(0, c), :], xbuf, sx).wait()
            pltpu.make_async_copy(w_hbm.at[pl.ds(0, c)], wbuf, sw).wait()

        def compute(xbuf, wbuf, acc):
            def grp(g, acc):
                wv = wbuf[pl.ds(g * 16, 16)]
                e = jnp.exp(wv - mv) * cv
                a0, a1, a2, a3 = acc
                for j in range(16):
                    ej = _splat16(e, jnp.full((16,), j, jnp.int32))
                    row = g * 16 + j
                    a0 = a0 + ej * xbuf[row, pl.ds(0, 16)]
                    a1 = a1 + ej * xbuf[row, pl.ds(16, 16)]
                    a2 = a2 + ej * xbuf[row, pl.ds(32, 16)]
                    a3 = a3 + ej * xbuf[row, pl.ds(48, 16)]
                return a0, a1, a2, a3
            return lax.fori_loop(0, c // 16, grp, acc)

        start_chunk(0, xa, wa, sxa, swa)
        start_chunk(1, xb, wb, sxb, swb)
        z = jnp.zeros((16,), jnp.float32)

        def body(i, acc):
            j0 = 2 * i
            wait_chunk(xa, wa, sxa, swa)
            acc = compute(xa, wa, acc)

            @pl.when(j0 + 2 < nc)
            def _():
                start_chunk(j0 + 2, xa, wa, sxa, swa)

            wait_chunk(xb, wb, sxb, swb)
            acc = compute(xb, wb, acc)

            @pl.when(j0 + 3 < nc)
            def _():
                start_chunk(j0 + 3, xb, wb, sxb, swb)

            return acc

        acc = lax.fori_loop(0, nc // 2, body, (z, z, z, z))
        ob[pl.ds(0, 16)] = acc[0]
        ob[pl.ds(16, 16)] = acc[1]
        ob[pl.ds(32, 16)] = acc[2]
        ob[pl.ds(48, 16)] = acc[3]

        @pl.when(cid == 0)
        def _w0():
            pltpu.sync_copy(ob, out0.at[pl.ds(sid * t, t)])

        @pl.when(cid == 1)
        def _w1():
            pltpu.sync_copy(ob, out1.at[pl.ds(sid * t, t)])

    return sc_fn


def _pick_block(n):
    for b in (8000, 10000, 5000, 4096, 4000, 2048, 2000, 1000):
        if n % b == 0:
            return b
    return n


def _tc_wsum(stats, w, x, first_block, num_blocks, b):
    n, t = x.shape
    g_total = n // b
    return pl.pallas_call(
        _wsum_tc_grid_kernel,
        grid=(num_blocks,),
        out_shape=jax.ShapeDtypeStruct((1, t), jnp.float32),
        in_specs=[
            pl.BlockSpec((2, 16), lambda i: (0, 0)),
            pl.BlockSpec((1, 1, b), lambda i: (first_block + i, 0, 0)),
            pl.BlockSpec((b, t), lambda i: (first_block + i, 0)),
        ],
        out_specs=pl.BlockSpec((1, t), lambda i: (0, 0)),
    )(stats, w.reshape(g_total, 1, b), x)


def kernel(x, w, k_param):
    n, t = x.shape
    b = _pick_block(n)
    rows = 1000 if n % 1000 == 0 else 1

    stats = pl.pallas_call(
        _stats_kernel,
        out_shape=jax.ShapeDtypeStruct((2, 16), jnp.float32),
        in_specs=[
            pl.BlockSpec((n // rows, rows), lambda: (0, 0)),
            pl.BlockSpec((1, 1), lambda: (0, 0)),
        ],
        out_specs=pl.BlockSpec((2, 16), lambda: (0, 0)),
    )(w.reshape(n // rows, rows), k_param.reshape(1, 1))

    stripe = _NUM_WORKERS * _CHUNK
    n_sc = (n * _SC_FRACTION_NUM // _SC_FRACTION_DEN) // stripe * stripe
    while n_sc > 0 and not ((n_sc // stripe) % 2 == 0
                            and (n - n_sc) % b == 0 and n_sc % b == 0):
        n_sc -= stripe
    use_sc = t == 64 and n_sc > 0

    if use_sc:
        sc0, sc1 = _make_sc_wsum(n_sc, t)(stats, w, x)
        tc_part = _tc_wsum(stats, w, x, n_sc // b, (n - n_sc) // b, b)
        out = pl.pallas_call(
            _combine_kernel,
            out_shape=jax.ShapeDtypeStruct((1, t), jnp.float32),
            in_specs=[
                pl.BlockSpec((1, t), lambda: (0, 0)),
                pl.BlockSpec((_NUM_SUBCORES, t), lambda: (0, 0)),
                pl.BlockSpec((_NUM_SUBCORES, t), lambda: (0, 0)),
            ],
            out_specs=pl.BlockSpec((1, t), lambda: (0, 0)),
        )(tc_part, sc0.reshape(_NUM_SUBCORES, t), sc1.reshape(_NUM_SUBCORES, t))
    else:
        out = _tc_wsum(stats, w, x, 0, n // b, b)

    return out.reshape(t)


# speedup vs baseline: 5.9208x; 5.6264x over previous
import jax
import jax.numpy as jnp
from jax.experimental import pallas as pl
from jax.experimental.pallas import tpu as pltpu

_COLS = 25600


def _stats_tail_kernel(w_ref, k_ref, wt_ref, xt_ref, stats_ref, tail_ref):
    wv = w_ref[...]
    m = jnp.max(wv)
    d = jnp.sum(jnp.exp(wv - m))
    coeff = jnp.round(k_ref[0, 0]) / (jnp.float32(wv.size) * d)
    stats_ref[...] = jnp.stack([jnp.full((16,), m), jnp.full((16,), coeff)])
    e_t = jnp.exp(wt_ref[...] - m) * coeff
    tail_ref[...] = jax.lax.dot_general(
        e_t, xt_ref[...], (((1,), (0,)), ((), ())),
        preferred_element_type=jnp.float32)


def _stats_kernel(w_ref, k_ref, out_ref):
    wv = w_ref[...]
    m = jnp.max(wv)
    d = jnp.sum(jnp.exp(wv - m))
    coeff = jnp.round(k_ref[0, 0]) / (jnp.float32(wv.size) * d)
    out_ref[...] = jnp.stack([jnp.full((16,), m), jnp.full((16,), coeff)])


def _wsum_tc_grid_kernel(stats_ref, w_ref, x_ref, out_ref):
    i = pl.program_id(0)
    m = stats_ref[0, 0]
    coeff = stats_ref[1, 0]
    e = jnp.exp(w_ref[0] - m) * coeff
    part = jax.lax.dot_general(
        e, x_ref[...], (((1,), (0,)), ((), ())),
        preferred_element_type=jnp.float32)

    @pl.when(i == 0)
    def _init():
        out_ref[...] = jnp.zeros_like(out_ref)

    out_ref[...] += part


def _make_xt_kernel(n, t, b, nb):
    def body(stats_ref, tailp_ref, w_ref, xt_ref, out_ref,
             xa, xb_, wa, wb, sb, tb, acc, sxa, sxb, swa, swb):
        pltpu.sync_copy(stats_ref, sb)
        pltpu.sync_copy(tailp_ref, tb)
        m = sb[0, 0]
        coeff = sb[1, 0]

        xbufs = (xa, xb_)
        wbufs = (wa, wb)
        sxs = (sxa, sxb)
        sws = (swa, swb)

        def start(j, p):
            pltpu.async_copy(xt_ref.at[:, pl.ds(j * b, b)], xbufs[p], sxs[p])
            pltpu.async_copy(w_ref.at[pl.ds(j * b, b)], wbufs[p], sws[p])

        def wait(p):
            pltpu.make_async_copy(
                xt_ref.at[:, pl.ds(0, b)], xbufs[p], sxs[p]).wait()
            pltpu.make_async_copy(
                w_ref.at[pl.ds(0, b)], wbufs[p], sws[p]).wait()

        def process(p):
            e = (jnp.exp(wbufs[p][...] - m) * coeff).reshape(1, b)
            acc[...] += xbufs[p][...] * e

        start(0, 0)
        if nb > 1:
            start(1, 1)
        acc[...] = jnp.zeros_like(acc)

        def pair(i, carry):
            j0 = 2 * i
            wait(0)
            process(0)

            @pl.when(j0 + 2 < nb)
            def _():
                start(j0 + 2, 0)

            wait(1)
            process(1)

            @pl.when(j0 + 3 < nb)
            def _():
                start(j0 + 3, 1)

            return carry

        jax.lax.fori_loop(0, nb // 2, pair, 0)
        if nb % 2 == 1:
            wait((nb - 1) % 2)
            process((nb - 1) % 2)
        out_ref[...] = jnp.sum(acc[...], axis=1).reshape(1, t) + tb[...]

    return pl.pallas_call(
        body,
        out_shape=jax.ShapeDtypeStruct((1, t), jnp.float32),
        in_specs=[
            pl.BlockSpec(memory_space=pltpu.HBM),
            pl.BlockSpec(memory_space=pltpu.HBM),
            pl.BlockSpec(memory_space=pltpu.HBM),
            pl.BlockSpec(memory_space=pltpu.HBM),
        ],
        out_specs=pl.BlockSpec((1, t), lambda: (0, 0)),
        scratch_shapes=[
            pltpu.VMEM((t, b), jnp.float32),
            pltpu.VMEM((t, b), jnp.float32),
            pltpu.VMEM((b,), jnp.float32),
            pltpu.VMEM((b,), jnp.float32),
            pltpu.VMEM((2, 16), jnp.float32),
            pltpu.VMEM((1, t), jnp.float32),
            pltpu.VMEM((t, b), jnp.float32),
            pltpu.SemaphoreType.DMA,
            pltpu.SemaphoreType.DMA,
            pltpu.SemaphoreType.DMA,
            pltpu.SemaphoreType.DMA,
        ],
    )


def _pick_block(n):
    for b in (8000, 10000, 5000, 4096, 4000, 2048, 2000, 1000):
        if n % b == 0:
            return b
    return n


def kernel(x, w, k_param):
    n, t = x.shape
    rows = 1000 if n % 1000 == 0 else 1
    w2d = w.reshape(n // rows, rows)
    k2d = k_param.reshape(1, 1)

    bc = _COLS
    nb = n // bc
    tail = n - nb * bc
    use_xt = (t % 8 == 0 and nb >= 2 and tail % 8 == 0 and tail > 0
              and bc % 1024 == 0)

    if use_xt:
        wt = w[n - tail:].reshape(1, tail)
        xtail = x[n - tail:]
        stats, tailp = pl.pallas_call(
            _stats_tail_kernel,
            out_shape=(
                jax.ShapeDtypeStruct((2, 16), jnp.float32),
                jax.ShapeDtypeStruct((1, t), jnp.float32),
            ),
            in_specs=[
                pl.BlockSpec((n // rows, rows), lambda: (0, 0)),
                pl.BlockSpec((1, 1), lambda: (0, 0)),
                pl.BlockSpec((1, tail), lambda: (0, 0)),
                pl.BlockSpec((tail, t), lambda: (0, 0)),
            ],
            out_specs=(
                pl.BlockSpec((2, 16), lambda: (0, 0)),
                pl.BlockSpec((1, t), lambda: (0, 0)),
            ),
        )(w2d, k2d, wt, xtail)
        out = _make_xt_kernel(n, t, bc, nb)(stats, tailp, w, x.T)
    else:
        stats = pl.pallas_call(
            _stats_kernel,
            out_shape=jax.ShapeDtypeStruct((2, 16), jnp.float32),
            in_specs=[
                pl.BlockSpec((n // rows, rows), lambda: (0, 0)),
                pl.BlockSpec((1, 1), lambda: (0, 0)),
            ],
            out_specs=pl.BlockSpec((2, 16), lambda: (0, 0)),
        )(w2d, k2d)
        b = _pick_block(n)
        out = pl.pallas_call(
            _wsum_tc_grid_kernel,
            grid=(n // b,),
            out_shape=jax.ShapeDtypeStruct((1, t), jnp.float32),
            in_specs=[
                pl.BlockSpec((2, 16), lambda i: (0, 0)),
                pl.BlockSpec((1, 1, b), lambda i: (i, 0, 0)),
                pl.BlockSpec((b, t), lambda i: (i, 0)),
            ],
            out_specs=pl.BlockSpec((1, t), lambda i: (0, 0)),
        )(stats, w.reshape(n // b, 1, b), x)

    return out.reshape(t)
